# baseline (speedup 1.0000x reference)
import functools

import jax
import jax.numpy as jnp
from jax import lax
from jax.experimental import pallas as pl
from jax.experimental.pallas import tpu as pltpu
from jax.experimental.pallas import tpu_sc as plsc

B = 16384
D = 128
NC = 2
NS = 16
NW = NC * NS
ROWS_PER_W = B // NW
CHUNK = 128
NCHUNK = ROWS_PER_W // CHUNK
LANES = 16


@functools.partial(
    pl.kernel,
    out_type=jax.ShapeDtypeStruct((NW, LANES), jnp.float32),
    mesh=plsc.VectorSubcoreMesh(core_axis_name="c", subcore_axis_name="s"),
    compiler_params=pltpu.CompilerParams(needs_layout_passes=False),
    scratch_types=(
        pltpu.VMEM((ROWS_PER_W,), jnp.int32),
        pltpu.VMEM((CHUNK, D), jnp.float32),
        pltpu.VMEM((CHUNK, D), jnp.float32),
        pltpu.VMEM((CHUNK, D), jnp.float32),
        pltpu.VMEM((CHUNK, D), jnp.float32),
        pltpu.VMEM((LANES,), jnp.float32),
        pltpu.SemaphoreType.DMA,
        pltpu.SemaphoreType.DMA,
    ),
)
def _sc_loss(f_hbm, lab_hbm, cen_hbm, part_hbm,
             lab_v, f0, f1, c0, c1, part_v, sem0, sem1):
    cid = lax.axis_index("c")
    sid = lax.axis_index("s")
    wid = sid * NC + cid
    base = wid * ROWS_PER_W

    pltpu.sync_copy(lab_hbm.at[pl.ds(base, ROWS_PER_W)], lab_v)

    fb = (f0, f1)
    cb = (c0, c1)
    sems = (sem0, sem1)
    NBUF = 2

    def start(ci):
        b = ci % NBUF
        g = pltpu.async_copy(
            cen_hbm.at[lab_v.at[pl.ds(ci * CHUNK, CHUNK)]], cb[b], sems[b])
        f = pltpu.async_copy(
            f_hbm.at[pl.ds(base + ci * CHUNK, CHUNK)], fb[b], sems[b])
        return (g, f)

    pend = [start(0)]
    lane = lax.iota(jnp.int32, LANES)
    loss = jnp.zeros((LANES,), jnp.float32)
    for ci in range(NCHUNK):
        for dsc in pend.pop(0):
            dsc.wait()
        if ci + 1 < NCHUNK:
            pend.append(start(ci + 1))
        f_v = fb[ci % NBUF]
        c_v = cb[ci % NBUF]
        for g in range(CHUNK // LANES):
            rows = g * LANES + lane

            def kbody(k, carry, rows=rows, f_v=f_v, c_v=c_v):
                acc_d, acc_f, acc_c = carry
                col = (k + lane) & (D - 1)
                fv = plsc.load_gather(f_v, [rows, col])
                cv = plsc.load_gather(c_v, [rows, col])
                return (acc_d + fv * cv, acc_f + fv * fv, acc_c + cv * cv)

            z = jnp.zeros((LANES,), jnp.float32)
            acc_d, acc_f, acc_c = lax.fori_loop(0, D, kbody, (z, z, z),
                                                unroll=8)
            prod = jnp.maximum(acc_f, 1e-24) * jnp.maximum(acc_c, 1e-24)
            i = plsc.bitcast(prod, jnp.int32)
            y = plsc.bitcast(0x5F3759DF - lax.shift_right_logical(i, 1),
                             jnp.float32)
            for _ in range(3):
                y = y * (1.5 - 0.5 * prod * y * y)
            sim = acc_d * y
            dist = jnp.clip(1.0 - sim, 0.0, 2.0)
            loss = loss + dist
    part_v[...] = loss
    pltpu.sync_copy(part_v, part_hbm.at[wid])


def _tc_finish(part_ref, out_ref):
    out_ref[0, 0] = jnp.sum(part_ref[...]) * (1.0 / B)


def kernel(features, labels, centers):
    parts = _sc_loss(features, labels, centers)
    return jnp.sum(parts) * (1.0 / B)


# speedup vs baseline: 1.2250x; 1.2250x over previous
import functools

import jax
import jax.numpy as jnp
from jax import lax
from jax.experimental import pallas as pl
from jax.experimental.pallas import tpu as pltpu
from jax.experimental.pallas import tpu_sc as plsc

B = 16384
D = 128
NC = 2
NS = 16
NW = NC * NS
ROWS_PER_W = B // NW
CHUNK = 128
NCHUNK = ROWS_PER_W // CHUNK
LANES = 16


@functools.partial(
    pl.kernel,
    out_type=jax.ShapeDtypeStruct((NW, LANES), jnp.float32),
    mesh=plsc.VectorSubcoreMesh(core_axis_name="c", subcore_axis_name="s"),
    compiler_params=pltpu.CompilerParams(needs_layout_passes=False),
    scratch_types=(
        pltpu.VMEM((ROWS_PER_W,), jnp.int32),
        pltpu.VMEM((CHUNK, D), jnp.float32),
        pltpu.VMEM((CHUNK, D), jnp.float32),
        pltpu.VMEM((CHUNK, D), jnp.float32),
        pltpu.VMEM((CHUNK, D), jnp.float32),
        pltpu.VMEM((LANES,), jnp.float32),
        pltpu.SemaphoreType.DMA,
        pltpu.SemaphoreType.DMA,
    ),
)
def _sc_loss(f_hbm, lab_hbm, cen_hbm, part_hbm,
             lab_v, f0, f1, c0, c1, part_v, sem0, sem1):
    cid = lax.axis_index("c")
    sid = lax.axis_index("s")
    wid = sid * NC + cid
    base = wid * ROWS_PER_W

    fb = (f0, f1)
    cb = (c0, c1)
    sems = (sem0, sem1)
    NBUF = 2

    def start_f(ci):
        b = ci % NBUF
        return pltpu.async_copy(
            f_hbm.at[pl.ds(base + ci * CHUNK, CHUNK)], fb[b], sems[b])

    def start_g(ci):
        b = ci % NBUF
        return pltpu.async_copy(
            cen_hbm.at[lab_v.at[pl.ds(ci * CHUNK, CHUNK)]], cb[b], sems[b])

    def start(ci):
        return (start_g(ci), start_f(ci))

    f0_dsc = start_f(0)
    pltpu.sync_copy(lab_hbm.at[pl.ds(base, ROWS_PER_W)], lab_v)
    pend = [(start_g(0), f0_dsc)]
    lane = lax.iota(jnp.int32, LANES)
    loss = jnp.zeros((LANES,), jnp.float32)
    for ci in range(NCHUNK):
        for dsc in pend.pop(0):
            dsc.wait()
        if ci + 1 < NCHUNK:
            pend.append(start(ci + 1))
        f_v = fb[ci % NBUF]
        c_v = cb[ci % NBUF]

        def gbody(g, loss, f_v=f_v, c_v=c_v):
            rows = g * LANES + lane

            def kbody(k, carry):
                acc_d, acc_f, acc_c = carry
                col = (k + lane) & (D - 1)
                fv = plsc.load_gather(f_v, [rows, col])
                cv = plsc.load_gather(c_v, [rows, col])
                return (acc_d + fv * cv, acc_f + fv * fv, acc_c + cv * cv)

            z = jnp.zeros((LANES,), jnp.float32)
            acc_d, acc_f, acc_c = lax.fori_loop(0, D, kbody, (z, z, z),
                                                unroll=4)
            prod = jnp.maximum(acc_f, 1e-24) * jnp.maximum(acc_c, 1e-24)
            i = plsc.bitcast(prod, jnp.int32)
            y = plsc.bitcast(0x5F3759DF - lax.shift_right_logical(i, 1),
                             jnp.float32)
            for _ in range(3):
                y = y * (1.5 - 0.5 * prod * y * y)
            sim = acc_d * y
            dist = jnp.clip(1.0 - sim, 0.0, 2.0)
            return loss + dist

        loss = lax.fori_loop(0, CHUNK // LANES, gbody, loss)
    part_v[...] = loss
    pltpu.sync_copy(part_v, part_hbm.at[wid])


def _tc_finish(part_ref, out_ref):
    out_ref[0, 0] = jnp.sum(part_ref[...]) * (1.0 / B)


def kernel(features, labels, centers):
    parts = _sc_loss(features, labels, centers)
    out = pl.pallas_call(
        _tc_finish,
        out_shape=jax.ShapeDtypeStruct((1, 1), jnp.float32),
        out_specs=pl.BlockSpec(memory_space=pltpu.SMEM),
    )(parts)
    return out[0, 0]
